# baseline (speedup 1.0000x reference)
import functools
import jax
import jax.numpy as jnp
from jax import lax
from jax.experimental import pallas as pl
from jax.experimental.pallas import tpu as pltpu
from jax.experimental.pallas import tpu_sc as plsc

NUM_BINS = 128
EMB_DIM = 64
NC = 2
NS = 16
NW = NC * NS

CHUNK = 128
PAIR = 512
NB = 2
LANES = 16


def _idx_body(num_ref, idx_ref):
    x = num_ref[...]
    l = jnp.log2(jnp.abs(x) + 1e-10)
    t = (l + 10.0) * 6.4
    k = jnp.clip(jnp.ceil(t).astype(jnp.int32), 0, 129)
    bk = -10.0 + k.astype(jnp.float32) * 0.15625
    bkm1 = -10.0 + (k - 1).astype(jnp.float32) * 0.15625
    k = jnp.where((k <= 128) & (bk < l), k + 1, k)
    k = jnp.where((k >= 1) & (bkm1 >= l), k - 1, k)
    b = jnp.clip(k, 0, 127)
    sgn = (x > 0.0).astype(jnp.int32) - (x < 0.0).astype(jnp.int32)
    idx_ref[...] = b + 128 * (sgn + 1)


def _table_body(w_ref, s_ref, out_ref):
    ws = w_ref[...] * s_ref[...]
    wst = ws.T
    out_ref[:, 0:NUM_BINS] = -wst
    out_ref[:, NUM_BINS:2 * NUM_BINS] = jnp.zeros_like(wst)
    out_ref[:, 2 * NUM_BINS:3 * NUM_BINS] = wst


def _sc_tgather(idx_hbm, w3t_hbm, out_hbm, idx_v, w3t_v, tbuf, wsem):
    wid = lax.axis_index("s") * NC + lax.axis_index("c")
    pltpu.sync_copy(w3t_hbm, w3t_v)
    pltpu.sync_copy(idx_hbm.at[wid], idx_v)

    ncb = out_hbm.shape[2]
    chunks_per_w = idx_v.shape[0] // PAIR
    gbase = wid * chunks_per_w

    def dst(g):
        gg = gbase + g
        return out_hbm.at[gg // ncb, :, gg % ncb]

    for j in range(NB):
        pltpu.async_copy(tbuf.at[j], dst(j), wsem)

    def group(g0, carry):
        for j in range(NB):
            g = g0 * NB + j
            pltpu.make_async_copy(tbuf.at[j], dst(g), wsem).wait()
            idx16 = [
                idx_v[pl.ds(g * PAIR + v * LANES, LANES)]
                for v in range(PAIR // LANES)
            ]

            @plsc.parallel_loop(0, EMB_DIM, unroll=2)
            def dloop(d):
                row0 = d * (3 * NUM_BINS)
                for v in range(PAIR // LANES):
                    vals = plsc.load_gather(w3t_v, [idx16[v] + row0])
                    off = (v // 8) * 1024 + (d % 8) * CHUNK + (v % 8) * LANES
                    tbuf[j, d // 8, pl.ds(off, LANES)] = vals

            pltpu.async_copy(tbuf.at[j], dst(g), wsem)
        return carry

    lax.fori_loop(0, chunks_per_w // NB, group, 0)

    for j in range(NB):
        pltpu.make_async_copy(tbuf.at[j], dst(j), wsem).wait()


def kernel(number, W, scale):
    squeeze = number.ndim == 1
    if squeeze:
        number = number[None, :]
    B, N = number.shape
    M = B * N
    assert M % (NW * CHUNK) == 0 and N % CHUNK == 0
    elems_per_w = M // NW

    rows_blk = max(8, min(B, (1 << 22) // (4 * N)))
    while B % rows_blk:
        rows_blk //= 2
    idx = pl.pallas_call(
        _idx_body,
        grid=(B // rows_blk,),
        in_specs=[pl.BlockSpec((rows_blk, N), lambda i: (i, 0))],
        out_specs=pl.BlockSpec((rows_blk, N), lambda i: (i, 0)),
        out_shape=jax.ShapeDtypeStruct((B, N), jnp.int32),
    )(number)

    w3t = pl.pallas_call(
        _table_body,
        out_shape=jax.ShapeDtypeStruct((EMB_DIM, 3 * NUM_BINS), jnp.float32),
    )(W, scale.reshape(NUM_BINS, 1))

    idx2 = idx.reshape(NW, elems_per_w)
    w3t_flat = w3t.reshape(EMB_DIM * 3 * NUM_BINS)

    mesh = plsc.VectorSubcoreMesh(
        core_axis_name="c", subcore_axis_name="s", num_cores=NC, num_subcores=NS
    )
    out5 = pl.kernel(
        _sc_tgather,
        out_type=jax.ShapeDtypeStruct(
            (B, EMB_DIM // 8, N // PAIR, 4 * 8 * CHUNK), jnp.float32
        ),
        mesh=mesh,
        scratch_types=[
            pltpu.VMEM((elems_per_w,), jnp.int32),
            pltpu.VMEM((EMB_DIM * 3 * NUM_BINS,), jnp.float32),
            pltpu.VMEM((NB, EMB_DIM // 8, 4 * 8 * CHUNK), jnp.float32),
            pltpu.SemaphoreType.DMA,
        ],
        compiler_params=pltpu.CompilerParams(use_tc_tiling_on_sc=False, needs_layout_passes=False),
    )(idx2, w3t_flat)

    out = (
        out5.reshape(B, EMB_DIM // 8, N // CHUNK, 8, CHUNK)
        .transpose(0, 2, 4, 1, 3)
        .reshape(B, N, EMB_DIM)
    )
    if squeeze:
        out = out[0]
    return out


# speedup vs baseline: 2.2219x; 2.2219x over previous
import functools
import jax
import jax.numpy as jnp
from jax import lax
from jax.experimental import pallas as pl
from jax.experimental.pallas import tpu as pltpu
from jax.experimental.pallas import tpu_sc as plsc

NUM_BINS = 128
EMB_DIM = 64
NC = 2
NS = 16
NW = NC * NS

CHUNK = 128
PAIR = 512
NB = 2
LANES = 16


def _idx_body(num_ref, idx_ref):
    x = num_ref[...]
    l = jnp.log2(jnp.abs(x) + 1e-10)
    t = (l + 10.0) * 6.4
    k = jnp.clip(jnp.ceil(t).astype(jnp.int32), 0, 129)
    bk = -10.0 + k.astype(jnp.float32) * 0.15625
    bkm1 = -10.0 + (k - 1).astype(jnp.float32) * 0.15625
    k = jnp.where((k <= 128) & (bk < l), k + 1, k)
    k = jnp.where((k >= 1) & (bkm1 >= l), k - 1, k)
    b = jnp.clip(k, 0, 127)
    sgn = (x > 0.0).astype(jnp.int32) - (x < 0.0).astype(jnp.int32)
    idx_ref[...] = b + 128 * (sgn + 1)


def _table_body(w_ref, s_ref, out_ref):
    ws = w_ref[...] * s_ref[...]
    wst = ws.T
    out_ref[:, 0:NUM_BINS] = -wst
    out_ref[:, NUM_BINS:2 * NUM_BINS] = jnp.zeros_like(wst)
    out_ref[:, 2 * NUM_BINS:3 * NUM_BINS] = wst


def _sc_tgather(idx_hbm, w3t_hbm, out_hbm, idx_v, w3t_v, tbuf, wsem):
    wid = lax.axis_index("s") * NC + lax.axis_index("c")
    pltpu.sync_copy(w3t_hbm, w3t_v)
    pltpu.sync_copy(idx_hbm.at[wid], idx_v)

    ncb = out_hbm.shape[2]
    chunks_per_w = idx_v.shape[0] // PAIR
    gbase = wid * chunks_per_w

    def dst(g):
        gg = gbase + g
        return out_hbm.at[gg // ncb, :, gg % ncb]

    for j in range(NB):
        pltpu.async_copy(tbuf.at[j], dst(j), wsem)

    def group(g0, carry):
        for j in range(NB):
            g = g0 * NB + j
            pltpu.make_async_copy(tbuf.at[j], dst(g), wsem).wait()
            idx16 = [
                idx_v[pl.ds(g * PAIR + v * LANES, LANES)]
                for v in range(PAIR // LANES)
            ]

            @plsc.parallel_loop(0, EMB_DIM, unroll=4)
            def dloop(d):
                row0 = d * (3 * NUM_BINS)
                for v in range(PAIR // LANES):
                    vals = plsc.load_gather(w3t_v, [idx16[v] + row0])
                    off = (v // 8) * 1024 + (d % 8) * CHUNK + (v % 8) * LANES
                    tbuf[j, d // 8, pl.ds(off, LANES)] = vals

            pltpu.async_copy(tbuf.at[j], dst(g), wsem)
        return carry

    lax.fori_loop(0, chunks_per_w // NB, group, 0)

    for j in range(NB):
        pltpu.make_async_copy(tbuf.at[j], dst(j), wsem).wait()


def kernel(number, W, scale):
    squeeze = number.ndim == 1
    if squeeze:
        number = number[None, :]
    B, N = number.shape
    M = B * N
    assert M % (NW * CHUNK) == 0 and N % CHUNK == 0
    elems_per_w = M // NW

    rows_blk = max(8, min(B, (1 << 22) // (4 * N)))
    while B % rows_blk:
        rows_blk //= 2
    idx = pl.pallas_call(
        _idx_body,
        grid=(B // rows_blk,),
        in_specs=[pl.BlockSpec((rows_blk, N), lambda i: (i, 0))],
        out_specs=pl.BlockSpec((rows_blk, N), lambda i: (i, 0)),
        out_shape=jax.ShapeDtypeStruct((B, N), jnp.int32),
    )(number)

    w3t = pl.pallas_call(
        _table_body,
        out_shape=jax.ShapeDtypeStruct((EMB_DIM, 3 * NUM_BINS), jnp.float32),
    )(W, scale.reshape(NUM_BINS, 1))

    idx2 = idx.reshape(NW, elems_per_w)
    w3t_flat = w3t.reshape(EMB_DIM * 3 * NUM_BINS)

    mesh = plsc.VectorSubcoreMesh(
        core_axis_name="c", subcore_axis_name="s", num_cores=NC, num_subcores=NS
    )
    out5 = pl.kernel(
        _sc_tgather,
        out_type=jax.ShapeDtypeStruct(
            (B, EMB_DIM // 8, N // PAIR, 4 * 8 * CHUNK), jnp.float32
        ),
        mesh=mesh,
        scratch_types=[
            pltpu.VMEM((elems_per_w,), jnp.int32),
            pltpu.VMEM((EMB_DIM * 3 * NUM_BINS,), jnp.float32),
            pltpu.VMEM((NB, EMB_DIM // 8, 4 * 8 * CHUNK), jnp.float32),
            pltpu.SemaphoreType.DMA,
        ],
        compiler_params=pltpu.CompilerParams(use_tc_tiling_on_sc=False, needs_layout_passes=False),
    )(idx2, w3t_flat)

    out = (
        out5.reshape(B, EMB_DIM // 8, N // CHUNK, 8, CHUNK)
        .transpose(0, 2, 4, 1, 3)
        .reshape(B, N, EMB_DIM)
    )
    if squeeze:
        out = out[0]
    return out
